# Initial kernel scaffold; baseline (speedup 1.0000x reference)
import functools

import jax
import jax.numpy as jnp
from jax import lax
from jax.experimental import pallas as pl
from jax.experimental.pallas import tpu as pltpu
from jax.experimental.pallas import tpu_sc as plsc

NLANE = 16
NUM_CORES = 2
NUM_SUBCORES = 16
NUM_WORKERS = NUM_CORES * NUM_SUBCORES
CHUNK = 64
EPS = 1e-12


def _rsqrt(v):
    i = plsc.bitcast(v, jnp.int32)
    i = jnp.int32(0x5F3759DF) - (i >> 1)
    y = plsc.bitcast(i, jnp.float32)
    half = jnp.full((NLANE,), 0.5, jnp.float32)
    three_half = jnp.full((NLANE,), 1.5, jnp.float32)
    hv = half * v
    for _ in range(3):
        y = y * (three_half - hv * y * y)
    return y


@functools.partial(jax.jit, static_argnums=(0, 1))
def _sc_embed_ln(n_tokens, hidden, word_ids, combo_ids, word_table,
                 combo_table, gamma, beta):
    ngrp = hidden // NLANE
    per_w = n_tokens // NUM_WORKERS
    n_chunks = per_w // CHUNK
    mesh = plsc.VectorSubcoreMesh(core_axis_name="c", subcore_axis_name="s")

    @functools.partial(
        pl.kernel,
        out_type=jax.ShapeDtypeStruct((n_tokens, hidden), jnp.float32),
        mesh=mesh,
        scratch_types=[
            pltpu.VMEM((CHUNK,), jnp.int32),
            pltpu.VMEM((CHUNK,), jnp.int32),
            pltpu.VMEM((CHUNK, hidden), jnp.float32),
            pltpu.VMEM((CHUNK, hidden), jnp.float32),
            pltpu.VMEM((hidden,), jnp.float32),
            pltpu.VMEM((hidden,), jnp.float32),
            pltpu.SemaphoreType.DMA,
            pltpu.SemaphoreType.DMA,
        ],
    )
    def k(wids_hbm, cids_hbm, wtab_hbm, ctab_hbm, gam_hbm, bet_hbm,
          out_hbm, idx_w, idx_c, buf_w, buf_c, gam_v, bet_v, sem_w, sem_c):
        wid = lax.axis_index("s") * NUM_CORES + lax.axis_index("c")
        base_w = wid * per_w
        pltpu.sync_copy(gam_hbm, gam_v)
        pltpu.sync_copy(bet_hbm, bet_v)

        inv_h = jnp.full((NLANE,), 1.0 / hidden, jnp.float32)
        eps_v = jnp.full((NLANE,), EPS, jnp.float32)

        def chunk_body(g, carry):
            base = base_w + g * CHUNK
            pltpu.sync_copy(wids_hbm.at[pl.ds(base, CHUNK)], idx_w)
            pltpu.sync_copy(cids_hbm.at[pl.ds(base, CHUNK)], idx_c)
            cp_w = pltpu.async_copy(wtab_hbm.at[idx_w], buf_w, sem_w)
            cp_c = pltpu.async_copy(ctab_hbm.at[idx_c], buf_c, sem_c)
            cp_w.wait()
            cp_c.wait()

            def tok_body(t, tc):
                s = jnp.zeros((NLANE,), jnp.float32)
                ss = jnp.zeros((NLANE,), jnp.float32)
                for j in range(ngrp):
                    sl = pl.ds(j * NLANE, NLANE)
                    x = buf_w[t, sl] + buf_c[t, sl]
                    buf_w[t, sl] = x
                    s = s + x
                    ss = ss + x * x
                mean = jnp.full((NLANE,), jnp.sum(s), jnp.float32) * inv_h
                m2 = jnp.full((NLANE,), jnp.sum(ss), jnp.float32) * inv_h
                var = m2 - mean * mean
                rstd = _rsqrt(var + eps_v)
                shift = mean * rstd
                for j in range(ngrp):
                    sl = pl.ds(j * NLANE, NLANE)
                    x = buf_w[t, sl]
                    y = x * rstd - shift
                    buf_w[t, sl] = y * gam_v[sl] + bet_v[sl]
                return tc

            lax.fori_loop(0, CHUNK, tok_body, 0)
            pltpu.sync_copy(buf_w, out_hbm.at[pl.ds(base, CHUNK)])
            return carry

        lax.fori_loop(0, n_chunks, chunk_body, 0)

    return k(word_ids, combo_ids, word_table, combo_table, gamma, beta)


def kernel(input_ids, position_ids, token_type_ids, word_table, pos_table,
           type_table, gamma, beta):
    b, s = input_ids.shape
    max_pos, hidden = pos_table.shape
    word_ids = input_ids.astype(jnp.int32).reshape(-1)
    combo_ids = (token_type_ids.astype(jnp.int32) * max_pos
                 + position_ids.astype(jnp.int32)).reshape(-1)
    combo_table = (type_table[:, None, :] + pos_table[None, :, :]).reshape(
        -1, hidden)
    out = _sc_embed_ln(b * s, hidden, word_ids, combo_ids, word_table,
                       combo_table, gamma, beta)
    return out.reshape(b, s, hidden)


# speedup vs baseline: 1.0158x; 1.0158x over previous
import functools

import jax
import jax.numpy as jnp
from jax import lax
from jax.experimental import pallas as pl
from jax.experimental.pallas import tpu as pltpu
from jax.experimental.pallas import tpu_sc as plsc

NLANE = 16
NUM_CORES = 2
NUM_SUBCORES = 16
NUM_WORKERS = NUM_CORES * NUM_SUBCORES
CHUNK = 64
EPS = 1e-12


def _allsum(x):
    lanes = lax.iota(jnp.int32, NLANE)
    for sh in (8, 4, 2, 1):
        perm = jnp.bitwise_xor(lanes, jnp.int32(sh))
        x = x + jnp.take_along_axis(x, perm, axis=0)
    return x


def _rsqrt(v):
    i = plsc.bitcast(v, jnp.int32)
    i = jnp.int32(0x5F3759DF) - (i >> 1)
    y = plsc.bitcast(i, jnp.float32)
    half = jnp.full((NLANE,), 0.5, jnp.float32)
    three_half = jnp.full((NLANE,), 1.5, jnp.float32)
    hv = half * v
    for _ in range(3):
        y = y * (three_half - hv * y * y)
    return y


@functools.partial(jax.jit, static_argnums=(0, 1))
def _sc_embed_ln(n_tokens, hidden, word_ids, combo_ids, word_table,
                 combo_table, gamma, beta):
    ngrp = hidden // NLANE
    per_w = n_tokens // NUM_WORKERS
    n_chunks = per_w // CHUNK
    mesh = plsc.VectorSubcoreMesh(core_axis_name="c", subcore_axis_name="s")

    @functools.partial(
        pl.kernel,
        out_type=jax.ShapeDtypeStruct((n_tokens, hidden), jnp.float32),
        mesh=mesh,
        compiler_params=pltpu.CompilerParams(needs_layout_passes=False),
        scratch_types=[
            pltpu.VMEM((CHUNK,), jnp.int32),
            pltpu.VMEM((CHUNK,), jnp.int32),
            pltpu.VMEM((CHUNK, hidden), jnp.float32),
            pltpu.VMEM((CHUNK, hidden), jnp.float32),
            pltpu.VMEM((hidden,), jnp.float32),
            pltpu.VMEM((hidden,), jnp.float32),
            pltpu.SemaphoreType.DMA,
            pltpu.SemaphoreType.DMA,
        ],
    )
    def k(wids_hbm, cids_hbm, wtab_hbm, ctab_hbm, gam_hbm, bet_hbm,
          out_hbm, idx_w, idx_c, buf_w, buf_c, gam_v, bet_v, sem_w, sem_c):
        wid = lax.axis_index("s") * NUM_CORES + lax.axis_index("c")
        base_w = wid * per_w
        pltpu.sync_copy(gam_hbm, gam_v)
        pltpu.sync_copy(bet_hbm, bet_v)

        inv_h = jnp.full((NLANE,), 1.0 / hidden, jnp.float32)
        eps_v = jnp.full((NLANE,), EPS, jnp.float32)

        def chunk_body(g, carry):
            base = base_w + g * CHUNK
            pltpu.sync_copy(wids_hbm.at[pl.ds(base, CHUNK)], idx_w)
            pltpu.sync_copy(cids_hbm.at[pl.ds(base, CHUNK)], idx_c)
            cp_w = pltpu.async_copy(wtab_hbm.at[idx_w], buf_w, sem_w)
            cp_c = pltpu.async_copy(ctab_hbm.at[idx_c], buf_c, sem_c)
            cp_w.wait()
            cp_c.wait()

            def tok_body(t, tc):
                s = jnp.zeros((NLANE,), jnp.float32)
                ss = jnp.zeros((NLANE,), jnp.float32)
                for j in range(ngrp):
                    sl = pl.ds(j * NLANE, NLANE)
                    x = buf_w[t, sl] + buf_c[t, sl]
                    buf_w[t, sl] = x
                    s = s + x
                    ss = ss + x * x
                mean = _allsum(s) * inv_h
                m2 = _allsum(ss) * inv_h
                var = m2 - mean * mean
                rstd = _rsqrt(var + eps_v)
                shift = mean * rstd
                for j in range(ngrp):
                    sl = pl.ds(j * NLANE, NLANE)
                    x = buf_w[t, sl]
                    y = x * rstd - shift
                    buf_w[t, sl] = y * gam_v[sl] + bet_v[sl]
                return tc

            lax.fori_loop(0, CHUNK, tok_body, 0)
            pltpu.sync_copy(buf_w, out_hbm.at[pl.ds(base, CHUNK)])
            return carry

        lax.fori_loop(0, n_chunks, chunk_body, 0)

    return k(word_ids, combo_ids, word_table, combo_table, gamma, beta)


def kernel(input_ids, position_ids, token_type_ids, word_table, pos_table,
           type_table, gamma, beta):
    b, s = input_ids.shape
    max_pos, hidden = pos_table.shape
    word_ids = input_ids.astype(jnp.int32).reshape(-1)
    combo_ids = (token_type_ids.astype(jnp.int32) * max_pos
                 + position_ids.astype(jnp.int32)).reshape(-1)
    combo_table = (type_table[:, None, :] + pos_table[None, :, :]).reshape(
        -1, hidden)
    out = _sc_embed_ln(b * s, hidden, word_ids, combo_ids, word_table,
                       combo_table, gamma, beta)
    return out.reshape(b, s, hidden)


# speedup vs baseline: 1.2830x; 1.2630x over previous
import functools

import jax
import jax.numpy as jnp
from jax import lax
from jax.experimental import pallas as pl
from jax.experimental.pallas import tpu as pltpu
from jax.experimental.pallas import tpu_sc as plsc

NLANE = 16
NUM_CORES = 2
NUM_SUBCORES = 16
NUM_WORKERS = NUM_CORES * NUM_SUBCORES
CHUNK = 64
EPS = 1e-12


def _allsum(x):
    lanes = lax.iota(jnp.int32, NLANE)
    for sh in (8, 4, 2, 1):
        perm = jnp.bitwise_xor(lanes, jnp.int32(sh))
        x = x + jnp.take_along_axis(x, perm, axis=0)
    return x


def _rsqrt(v):
    i = plsc.bitcast(v, jnp.int32)
    i = jnp.int32(0x5F3759DF) - (i >> 1)
    y = plsc.bitcast(i, jnp.float32)
    half = jnp.full((NLANE,), 0.5, jnp.float32)
    three_half = jnp.full((NLANE,), 1.5, jnp.float32)
    hv = half * v
    for _ in range(3):
        y = y * (three_half - hv * y * y)
    return y


@functools.partial(jax.jit, static_argnums=(0, 1))
def _sc_embed_ln(n_tokens, hidden, word_ids, combo_ids, word_table,
                 combo_table, gamma, beta):
    ngrp = hidden // NLANE
    per_w = n_tokens // NUM_WORKERS
    n_chunks = per_w // CHUNK
    mesh = plsc.VectorSubcoreMesh(core_axis_name="c", subcore_axis_name="s")

    @functools.partial(
        pl.kernel,
        out_type=jax.ShapeDtypeStruct((n_tokens, hidden), jnp.float32),
        mesh=mesh,
        compiler_params=pltpu.CompilerParams(needs_layout_passes=False),
        scratch_types=[
            pltpu.VMEM((CHUNK,), jnp.int32),
            pltpu.VMEM((CHUNK,), jnp.int32),
            pltpu.VMEM((CHUNK, hidden), jnp.float32),
            pltpu.VMEM((CHUNK, hidden), jnp.float32),
            pltpu.VMEM((hidden,), jnp.float32),
            pltpu.VMEM((hidden,), jnp.float32),
            pltpu.SemaphoreType.DMA,
            pltpu.SemaphoreType.DMA,
        ],
    )
    def k(wids_hbm, cids_hbm, wtab_hbm, ctab_hbm, gam_hbm, bet_hbm,
          out_hbm, idx_w, idx_c, buf_w, buf_c, gam_v, bet_v, sem_w, sem_c):
        wid = lax.axis_index("s") * NUM_CORES + lax.axis_index("c")
        base_w = wid * per_w
        pltpu.sync_copy(gam_hbm, gam_v)
        pltpu.sync_copy(bet_hbm, bet_v)

        inv_h = jnp.full((NLANE,), 1.0 / hidden, jnp.float32)
        eps_v = jnp.full((NLANE,), EPS, jnp.float32)

        def chunk_body(g, carry):
            base = base_w + g * CHUNK
            pltpu.sync_copy(wids_hbm.at[pl.ds(base, CHUNK)], idx_w)
            pltpu.sync_copy(cids_hbm.at[pl.ds(base, CHUNK)], idx_c)
            cp_w = pltpu.async_copy(wtab_hbm.at[idx_w], buf_w, sem_w)
            cp_c = pltpu.async_copy(ctab_hbm.at[idx_c], buf_c, sem_c)
            cp_w.wait()
            cp_c.wait()

            @plsc.parallel_loop(0, CHUNK)
            def tok_body(t):
                zeros = jnp.zeros((NLANE,), jnp.float32)
                acc_s = [zeros] * 4
                acc_q = [zeros] * 4
                xs = []
                for j in range(ngrp):
                    sl = pl.ds(j * NLANE, NLANE)
                    x = buf_w[t, sl] + buf_c[t, sl]
                    xs.append(x)
                    acc_s[j % 4] = acc_s[j % 4] + x
                    acc_q[j % 4] = acc_q[j % 4] + x * x
                s = (acc_s[0] + acc_s[1]) + (acc_s[2] + acc_s[3])
                q = (acc_q[0] + acc_q[1]) + (acc_q[2] + acc_q[3])
                mean = _allsum(s) * inv_h
                m2 = _allsum(q) * inv_h
                var = m2 - mean * mean
                rstd = _rsqrt(var + eps_v)
                shift = mean * rstd
                for j in range(ngrp):
                    sl = pl.ds(j * NLANE, NLANE)
                    y = xs[j] * rstd - shift
                    buf_w[t, sl] = y * gam_v[sl] + bet_v[sl]
            pltpu.sync_copy(buf_w, out_hbm.at[pl.ds(base, CHUNK)])
            return carry

        lax.fori_loop(0, n_chunks, chunk_body, 0)

    return k(word_ids, combo_ids, word_table, combo_table, gamma, beta)


def kernel(input_ids, position_ids, token_type_ids, word_table, pos_table,
           type_table, gamma, beta):
    b, s = input_ids.shape
    max_pos, hidden = pos_table.shape
    word_ids = input_ids.astype(jnp.int32).reshape(-1)
    combo_ids = (token_type_ids.astype(jnp.int32) * max_pos
                 + position_ids.astype(jnp.int32)).reshape(-1)
    combo_table = (type_table[:, None, :] + pos_table[None, :, :]).reshape(
        -1, hidden)
    out = _sc_embed_ln(b * s, hidden, word_ids, combo_ids, word_table,
                       combo_table, gamma, beta)
    return out.reshape(b, s, hidden)


# speedup vs baseline: 1.4689x; 1.1449x over previous
import functools

import jax
import jax.numpy as jnp
from jax import lax
from jax.experimental import pallas as pl
from jax.experimental.pallas import tpu as pltpu
from jax.experimental.pallas import tpu_sc as plsc

NLANE = 16
NUM_CORES = 2
NUM_SUBCORES = 16
NUM_WORKERS = NUM_CORES * NUM_SUBCORES
CHUNK = 32
EPS = 1e-12


def _allsum(x):
    lanes = lax.iota(jnp.int32, NLANE)
    for sh in (8, 4, 2, 1):
        perm = jnp.bitwise_xor(lanes, jnp.int32(sh))
        x = x + jnp.take_along_axis(x, perm, axis=0)
    return x


def _rsqrt(v):
    i = plsc.bitcast(v, jnp.int32)
    i = jnp.int32(0x5F3759DF) - (i >> 1)
    y = plsc.bitcast(i, jnp.float32)
    half = jnp.full((NLANE,), 0.5, jnp.float32)
    three_half = jnp.full((NLANE,), 1.5, jnp.float32)
    hv = half * v
    for _ in range(3):
        y = y * (three_half - hv * y * y)
    return y


@functools.partial(jax.jit, static_argnums=(0, 1))
def _sc_embed_ln(n_tokens, hidden, word_ids, combo_ids, word_table,
                 combo_table, gamma, beta):
    ngrp = hidden // NLANE
    per_w = n_tokens // NUM_WORKERS
    n_chunks = per_w // CHUNK
    mesh = plsc.VectorSubcoreMesh(core_axis_name="c", subcore_axis_name="s")

    @functools.partial(
        pl.kernel,
        out_type=jax.ShapeDtypeStruct((n_tokens, hidden), jnp.float32),
        mesh=mesh,
        compiler_params=pltpu.CompilerParams(needs_layout_passes=False),
        scratch_types=[
            pltpu.VMEM((per_w,), jnp.int32),
            pltpu.VMEM((per_w,), jnp.int32),
            pltpu.VMEM((2, CHUNK, hidden), jnp.float32),
            pltpu.VMEM((2, CHUNK, hidden), jnp.float32),
            pltpu.VMEM((hidden,), jnp.float32),
            pltpu.VMEM((hidden,), jnp.float32),
            pltpu.SemaphoreType.DMA,
            pltpu.SemaphoreType.DMA,
            pltpu.SemaphoreType.DMA,
            pltpu.SemaphoreType.DMA,
        ],
    )
    def k(wids_hbm, cids_hbm, wtab_hbm, ctab_hbm, gam_hbm, bet_hbm,
          out_hbm, idx_w, idx_c, buf_w, buf_c, gam_v, bet_v,
          sem_w0, sem_c0, sem_w1, sem_c1):
        wid = lax.axis_index("s") * NUM_CORES + lax.axis_index("c")
        base_w = wid * per_w
        pltpu.sync_copy(gam_hbm, gam_v)
        pltpu.sync_copy(bet_hbm, bet_v)
        pltpu.sync_copy(wids_hbm.at[pl.ds(base_w, per_w)], idx_w)
        pltpu.sync_copy(cids_hbm.at[pl.ds(base_w, per_w)], idx_c)

        inv_h = jnp.full((NLANE,), 1.0 / hidden, jnp.float32)
        eps_v = jnp.full((NLANE,), EPS, jnp.float32)
        sems = ((sem_w0, sem_c0), (sem_w1, sem_c1))

        def issue(g, slot):
            sw, sc = sems[slot]
            off = g * CHUNK
            pltpu.async_copy(
                wtab_hbm.at[idx_w.at[pl.ds(off, CHUNK)]], buf_w.at[slot], sw)
            pltpu.async_copy(
                ctab_hbm.at[idx_c.at[pl.ds(off, CHUNK)]], buf_c.at[slot], sc)

        def process(g, slot):
            sw, sc = sems[slot]
            pltpu.make_async_copy(
                wtab_hbm.at[idx_w.at[pl.ds(0, CHUNK)]], buf_w.at[slot],
                sw).wait()
            pltpu.make_async_copy(
                ctab_hbm.at[idx_c.at[pl.ds(0, CHUNK)]], buf_c.at[slot],
                sc).wait()

            @plsc.parallel_loop(0, CHUNK)
            def tok_body(t):
                zeros = jnp.zeros((NLANE,), jnp.float32)
                acc_s = [zeros] * 4
                acc_q = [zeros] * 4
                xs = []
                for j in range(ngrp):
                    sl = pl.ds(j * NLANE, NLANE)
                    x = buf_w[slot, t, sl] + buf_c[slot, t, sl]
                    xs.append(x)
                    acc_s[j % 4] = acc_s[j % 4] + x
                    acc_q[j % 4] = acc_q[j % 4] + x * x
                s = (acc_s[0] + acc_s[1]) + (acc_s[2] + acc_s[3])
                q = (acc_q[0] + acc_q[1]) + (acc_q[2] + acc_q[3])
                mean = _allsum(s) * inv_h
                m2 = _allsum(q) * inv_h
                var = m2 - mean * mean
                rstd = _rsqrt(var + eps_v)
                shift = mean * rstd
                for j in range(ngrp):
                    sl = pl.ds(j * NLANE, NLANE)
                    y = xs[j] * rstd - shift
                    buf_w[slot, t, sl] = y * gam_v[sl] + bet_v[sl]

            pltpu.sync_copy(buf_w.at[slot],
                            out_hbm.at[pl.ds(base_w + g * CHUNK, CHUNK)])

        issue(0, 0)

        def pair_body(i, carry):
            ga = 2 * i
            issue(ga + 1, 1)
            process(ga, 0)

            @pl.when(ga + 2 < n_chunks)
            def _():
                issue(ga + 2, 0)

            process(ga + 1, 1)
            return carry

        lax.fori_loop(0, n_chunks // 2, pair_body, 0)

    return k(word_ids, combo_ids, word_table, combo_table, gamma, beta)


def kernel(input_ids, position_ids, token_type_ids, word_table, pos_table,
           type_table, gamma, beta):
    b, s = input_ids.shape
    max_pos, hidden = pos_table.shape
    word_ids = input_ids.astype(jnp.int32).reshape(-1)
    combo_ids = (token_type_ids.astype(jnp.int32) * max_pos
                 + position_ids.astype(jnp.int32)).reshape(-1)
    combo_table = (type_table[:, None, :] + pos_table[None, :, :]).reshape(
        -1, hidden)
    out = _sc_embed_ln(b * s, hidden, word_ids, combo_ids, word_table,
                       combo_table, gamma, beta)
    return out.reshape(b, s, hidden)


# speedup vs baseline: 1.6800x; 1.1437x over previous
import functools

import jax
import jax.numpy as jnp
from jax import lax
from jax.experimental import pallas as pl
from jax.experimental.pallas import tpu as pltpu
from jax.experimental.pallas import tpu_sc as plsc

NLANE = 16
NUM_CORES = 2
NUM_SUBCORES = 16
NUM_WORKERS = NUM_CORES * NUM_SUBCORES
CHUNK = 32
EPS = 1e-12


def _allsum(x):
    lanes = lax.iota(jnp.int32, NLANE)
    for sh in (8, 4, 2, 1):
        perm = jnp.bitwise_xor(lanes, jnp.int32(sh))
        x = x + jnp.take_along_axis(x, perm, axis=0)
    return x


def _rsqrt(v):
    i = plsc.bitcast(v, jnp.int32)
    i = jnp.int32(0x5F3759DF) - (i >> 1)
    y = plsc.bitcast(i, jnp.float32)
    half = jnp.full((NLANE,), 0.5, jnp.float32)
    three_half = jnp.full((NLANE,), 1.5, jnp.float32)
    hv = half * v
    for _ in range(3):
        y = y * (three_half - hv * y * y)
    return y


@functools.partial(jax.jit, static_argnums=(0, 1))
def _sc_embed_ln(n_tokens, hidden, word_ids, combo_ids, word_table,
                 combo_table, gamma, beta):
    ngrp = hidden // NLANE
    per_w = n_tokens // NUM_WORKERS
    n_chunks = per_w // CHUNK
    mesh = plsc.VectorSubcoreMesh(core_axis_name="c", subcore_axis_name="s")

    @functools.partial(
        pl.kernel,
        out_type=jax.ShapeDtypeStruct((n_tokens, hidden), jnp.float32),
        mesh=mesh,
        compiler_params=pltpu.CompilerParams(needs_layout_passes=False),
        scratch_types=[
            pltpu.VMEM((per_w,), jnp.int32),
            pltpu.VMEM((per_w,), jnp.int32),
            pltpu.VMEM((2, CHUNK, hidden), jnp.float32),
            pltpu.VMEM((2, CHUNK, hidden), jnp.float32),
            pltpu.VMEM((hidden,), jnp.float32),
            pltpu.VMEM((hidden,), jnp.float32),
            pltpu.SemaphoreType.DMA,
            pltpu.SemaphoreType.DMA,
            pltpu.SemaphoreType.DMA,
            pltpu.SemaphoreType.DMA,
        ],
    )
    def k(wids_hbm, cids_hbm, wtab_hbm, ctab_hbm, gam_hbm, bet_hbm,
          out_hbm, idx_w, idx_c, buf_w, buf_c, gam_v, bet_v,
          sem_w0, sem_c0, sem_w1, sem_c1):
        wid = lax.axis_index("s") * NUM_CORES + lax.axis_index("c")
        base_w = wid * per_w
        pltpu.sync_copy(gam_hbm, gam_v)
        pltpu.sync_copy(bet_hbm, bet_v)
        pltpu.sync_copy(wids_hbm.at[pl.ds(base_w, per_w)], idx_w)
        pltpu.sync_copy(cids_hbm.at[pl.ds(base_w, per_w)], idx_c)

        inv_h = jnp.full((NLANE,), 1.0 / hidden, jnp.float32)
        eps_v = jnp.full((NLANE,), EPS, jnp.float32)
        sems = ((sem_w0, sem_c0), (sem_w1, sem_c1))

        def issue(g, slot):
            sw, sc = sems[slot]
            off = g * CHUNK
            pltpu.async_copy(
                wtab_hbm.at[idx_w.at[pl.ds(off, CHUNK)]], buf_w.at[slot], sw)
            pltpu.async_copy(
                ctab_hbm.at[idx_c.at[pl.ds(off, CHUNK)]], buf_c.at[slot], sc)

        def process(g, slot):
            sw, sc = sems[slot]
            pltpu.make_async_copy(
                wtab_hbm.at[idx_w.at[pl.ds(0, CHUNK)]], buf_w.at[slot],
                sw).wait()
            pltpu.make_async_copy(
                ctab_hbm.at[idx_c.at[pl.ds(0, CHUNK)]], buf_c.at[slot],
                sc).wait()

            @plsc.parallel_loop(0, CHUNK)
            def tok_body(t):
                zeros = jnp.zeros((NLANE,), jnp.float32)
                acc_s = [zeros] * 4
                acc_q = [zeros] * 4
                for j in range(ngrp):
                    sl = pl.ds(j * NLANE, NLANE)
                    x = buf_w[slot, t, sl] + buf_c[slot, t, sl]
                    buf_w[slot, t, sl] = x
                    acc_s[j % 4] = acc_s[j % 4] + x
                    acc_q[j % 4] = acc_q[j % 4] + x * x
                s = (acc_s[0] + acc_s[1]) + (acc_s[2] + acc_s[3])
                q = (acc_q[0] + acc_q[1]) + (acc_q[2] + acc_q[3])
                mean = _allsum(s) * inv_h
                m2 = _allsum(q) * inv_h
                var = m2 - mean * mean
                rstd = _rsqrt(var + eps_v)
                shift = mean * rstd
                for j in range(ngrp):
                    sl = pl.ds(j * NLANE, NLANE)
                    y = buf_w[slot, t, sl] * rstd - shift
                    buf_w[slot, t, sl] = y * gam_v[sl] + bet_v[sl]

            pltpu.sync_copy(buf_w.at[slot],
                            out_hbm.at[pl.ds(base_w + g * CHUNK, CHUNK)])

        issue(0, 0)

        def pair_body(i, carry):
            ga = 2 * i
            issue(ga + 1, 1)
            process(ga, 0)

            @pl.when(ga + 2 < n_chunks)
            def _():
                issue(ga + 2, 0)

            process(ga + 1, 1)
            return carry

        lax.fori_loop(0, n_chunks // 2, pair_body, 0)

    return k(word_ids, combo_ids, word_table, combo_table, gamma, beta)


def kernel(input_ids, position_ids, token_type_ids, word_table, pos_table,
           type_table, gamma, beta):
    b, s = input_ids.shape
    max_pos, hidden = pos_table.shape
    word_ids = input_ids.astype(jnp.int32).reshape(-1)
    combo_ids = (token_type_ids.astype(jnp.int32) * max_pos
                 + position_ids.astype(jnp.int32)).reshape(-1)
    combo_table = (type_table[:, None, :] + pos_table[None, :, :]).reshape(
        -1, hidden)
    out = _sc_embed_ln(b * s, hidden, word_ids, combo_ids, word_table,
                       combo_table, gamma, beta)
    return out.reshape(b, s, hidden)
